# baseline (speedup 1.0000x reference)
import jax
import jax.numpy as jnp
from jax.experimental import pallas as pl
from jax.experimental.pallas import tpu as pltpu

HIDDEN = 1024
FMIN = 0.1
FMAX = 0.5

B = 128
F = 32768
RC = 16
NCHUNK = B // RC
NBUF = 4


def _chunk_copy(x_hbm, buf, sems, c):
    return pltpu.make_async_copy(
        x_hbm.at[pl.ds(c * RC, RC), :], buf, sems.at[c % NBUF]
    )


def _psd_peak_body(x_hbm, out_ref, *scratch):
    bufs, sems = scratch[:NBUF], scratch[NBUF]

    for c in range(NBUF):
        _chunk_copy(x_hbm, bufs[c], sems, c).start()

    for c in range(NCHUNK):
        buf = bufs[c % NBUF]
        _chunk_copy(x_hbm, buf, sems, c).wait()

        blk = buf[...]
        bmax = jnp.max(blk, axis=1, keepdims=True)
        iota = jax.lax.broadcasted_iota(jnp.int32, (RC, F), 1)
        cand = jnp.where(blk == bmax, iota, F)
        peak = jnp.min(cand, axis=1, keepdims=True)

        idxf = peak.astype(jnp.float32)
        freq = FMIN + (FMAX - FMIN) * idxf / (F - 1)
        rr = freq * 60.0
        out_ref[pl.ds(c * RC, RC), :] = jnp.broadcast_to(rr, (RC, HIDDEN))

        if c + NBUF < NCHUNK:
            _chunk_copy(x_hbm, buf, sems, c + NBUF).start()


_psd_peak = pl.pallas_call(
    _psd_peak_body,
    in_specs=[pl.BlockSpec(memory_space=pl.ANY)],
    out_specs=pl.BlockSpec((B, HIDDEN), memory_space=pltpu.MemorySpace.VMEM),
    out_shape=jax.ShapeDtypeStruct((B, HIDDEN), jnp.float32),
    scratch_shapes=[pltpu.VMEM((RC, F), jnp.float32) for _ in range(NBUF)]
    + [pltpu.SemaphoreType.DMA((NBUF,))],
)


def kernel(x):
    return _psd_peak(x)


# speedup vs baseline: 1.0960x; 1.0960x over previous
import jax
import jax.numpy as jnp
from jax.experimental import pallas as pl
from jax.experimental.pallas import tpu as pltpu

HIDDEN = 1024
FMIN = 0.1
FMAX = 0.5

B = 128
F = 32768
RC = 32
NCHUNK = B // RC
NBUF = 4


def _chunk_copy(x_hbm, buf, sems, c):
    return pltpu.make_async_copy(
        x_hbm.at[pl.ds(c * RC, RC), :], buf, sems.at[c % NBUF]
    )


def _psd_peak_body(x_hbm, out_ref, *scratch):
    bufs, sems = scratch[:NBUF], scratch[NBUF]

    for c in range(NBUF):
        _chunk_copy(x_hbm, bufs[c], sems, c).start()

    for c in range(NCHUNK):
        buf = bufs[c % NBUF]
        _chunk_copy(x_hbm, buf, sems, c).wait()

        blk = buf[...]
        bmax = jnp.max(blk, axis=1, keepdims=True)
        iota = jax.lax.broadcasted_iota(jnp.int32, (RC, F), 1)
        cand = jnp.where(blk == bmax, iota, F)
        peak = jnp.min(cand, axis=1, keepdims=True)

        idxf = peak.astype(jnp.float32)
        freq = FMIN + (FMAX - FMIN) * idxf / (F - 1)
        rr = freq * 60.0
        out_ref[pl.ds(c * RC, RC), :] = jnp.broadcast_to(rr, (RC, HIDDEN))

        if c + NBUF < NCHUNK:
            _chunk_copy(x_hbm, buf, sems, c + NBUF).start()


_psd_peak = pl.pallas_call(
    _psd_peak_body,
    in_specs=[pl.BlockSpec(memory_space=pl.ANY)],
    out_specs=pl.BlockSpec((B, HIDDEN), memory_space=pltpu.MemorySpace.VMEM),
    out_shape=jax.ShapeDtypeStruct((B, HIDDEN), jnp.float32),
    scratch_shapes=[pltpu.VMEM((RC, F), jnp.float32) for _ in range(NBUF)]
    + [pltpu.SemaphoreType.DMA((NBUF,))],
)


def kernel(x):
    return _psd_peak(x)
